# baseline (speedup 1.0000x reference)
import functools

import jax
import jax.numpy as jnp
from jax.experimental import pallas as pl
from jax.experimental.pallas import tpu as pltpu


def _conv_pool_kernel(x_ref, w_ref, b_ref, o_ref, *, h, w, cout):
    xb = x_ref[0]
    wide = jnp.concatenate(
        [xb[:, dx:dx + w, :] for dx in range(3)], axis=2)
    patches = jnp.concatenate(
        [wide[dy:dy + h] for dy in range(3)], axis=2)
    p2 = patches.reshape(h * w, 27)
    y = jnp.dot(p2, w_ref[...], preferred_element_type=jnp.float32)
    y = jnp.maximum(y + b_ref[...], 0.0)
    y3 = y.reshape(h // 2, 2 * w, cout)
    rm = jnp.maximum(y3[:, :w, :], y3[:, w:, :])
    pm = jnp.maximum(rm[:, :w - 1, :], rm[:, 1:, :])
    col = jax.lax.broadcasted_iota(jnp.int32, (1, w - 1, 1), 1)
    s = jnp.sum(jnp.where(col % 2 == 0, pm, 0.0), axis=(0, 1))
    inv_area = 1.0 / ((h // 2) * (w // 2))
    o_ref[...] = (s * inv_area).reshape(1, 1, cout).astype(o_ref.dtype)


def _conv_pool_gap(x_nchw, conv_w, conv_b):
    n, cin, h, w = x_nchw.shape
    cout = conv_w.shape[0]
    x_nhwc = jnp.transpose(x_nchw.astype(jnp.bfloat16), (0, 2, 3, 1))
    xpad = jnp.pad(x_nhwc, ((0, 0), (1, 1), (1, 1), (0, 0)))
    w2 = jnp.transpose(conv_w, (2, 3, 1, 0)).reshape(9 * cin, cout)
    w2 = w2.astype(jnp.bfloat16)
    b2 = conv_b.reshape(1, cout)

    k = 9 * cin
    cost = pl.CostEstimate(
        flops=2 * n * h * w * k * cout,
        transcendentals=0,
        bytes_accessed=2 * n * (h + 2) * (w + 2) * cin + 4 * n * cout,
    )
    out = pl.pallas_call(
        functools.partial(_conv_pool_kernel, h=h, w=w, cout=cout),
        out_shape=jax.ShapeDtypeStruct((n, 1, cout), jnp.float32),
        grid=(n,),
        in_specs=[
            pl.BlockSpec((1, h + 2, w + 2, cin), lambda b: (b, 0, 0, 0)),
            pl.BlockSpec((k, cout), lambda b: (0, 0)),
            pl.BlockSpec((1, cout), lambda b: (0, 0)),
        ],
        out_specs=pl.BlockSpec((1, 1, cout), lambda b: (b, 0, 0)),
        compiler_params=pltpu.CompilerParams(
            dimension_semantics=("parallel",)
        ),
        cost_estimate=cost,
    )(xpad, w2, b2)
    return out.reshape(n, cout)


def _mlp_kernel(x_ref, w1_ref, b1_ref, w2_ref, b2_ref, w3_ref, b3_ref, o_ref):
    h1 = jnp.dot(x_ref[...], w1_ref[...], preferred_element_type=jnp.float32)
    h1 = jnp.maximum(h1 + b1_ref[...], 0.0).astype(jnp.bfloat16)
    h2 = jnp.dot(h1, w2_ref[...], preferred_element_type=jnp.float32)
    h2 = jnp.maximum(h2 + b2_ref[...], 0.0).astype(jnp.bfloat16)
    h3 = jnp.dot(h2, w3_ref[...], preferred_element_type=jnp.float32)
    o_ref[...] = (h3 + b3_ref[...]).astype(o_ref.dtype)


def _classifier(x, fc1_w, fc1_b, fc2_w, fc2_b, fc3_w, fc3_b):
    m, d = x.shape
    nout = fc3_w.shape[1]
    mb = m // 2
    return pl.pallas_call(
        _mlp_kernel,
        out_shape=jax.ShapeDtypeStruct((m, nout), jnp.float32),
        grid=(2,),
        in_specs=[
            pl.BlockSpec((mb, d), lambda i: (i, 0)),
            pl.BlockSpec((d, d), lambda i: (0, 0)),
            pl.BlockSpec((1, d), lambda i: (0, 0)),
            pl.BlockSpec((d, d), lambda i: (0, 0)),
            pl.BlockSpec((1, d), lambda i: (0, 0)),
            pl.BlockSpec((d, nout), lambda i: (0, 0)),
            pl.BlockSpec((1, nout), lambda i: (0, 0)),
        ],
        out_specs=pl.BlockSpec((mb, nout), lambda i: (i, 0)),
        compiler_params=pltpu.CompilerParams(
            dimension_semantics=("parallel",)
        ),
    )(
        x.astype(jnp.bfloat16),
        fc1_w.astype(jnp.bfloat16), fc1_b.reshape(1, -1),
        fc2_w.astype(jnp.bfloat16), fc2_b.reshape(1, -1),
        fc3_w.astype(jnp.bfloat16), fc3_b.reshape(1, -1),
    )


def kernel(x_nchw, conv_w, conv_b, fc1_w, fc1_b, fc2_w, fc2_b, fc3_w, fc3_b):
    pooled = _conv_pool_gap(x_nchw, conv_w, conv_b)
    return _classifier(pooled, fc1_w, fc1_b, fc2_w, fc2_b, fc3_w, fc3_b)


# speedup vs baseline: 2.1444x; 2.1444x over previous
import functools

import jax
import jax.numpy as jnp
from jax.experimental import pallas as pl
from jax.experimental.pallas import tpu as pltpu


def _conv_pool_kernel(x_ref, w_ref, b_ref, o_ref, *, h, w, cout):
    xt = jnp.transpose(x_ref[0], (1, 2, 0))
    xb = jnp.pad(xt, ((1, 1), (1, 1), (0, 0)))
    wide = jnp.concatenate(
        [xb[:, dx:dx + w, :] for dx in range(3)], axis=2)
    patches = jnp.concatenate(
        [wide[dy:dy + h] for dy in range(3)], axis=2)
    p2 = patches.reshape(h * w, 27)
    y = jnp.dot(p2, w_ref[...], preferred_element_type=jnp.float32)
    y = jnp.maximum(y + b_ref[...], 0.0)
    y3 = y.reshape(h // 2, 2 * w, cout)
    rm = jnp.maximum(y3[:, :w, :], y3[:, w:, :])
    pm = jnp.maximum(rm[:, :w - 1, :], rm[:, 1:, :])
    col = jax.lax.broadcasted_iota(jnp.int32, (1, w - 1, 1), 1)
    s = jnp.sum(jnp.where(col % 2 == 0, pm, 0.0), axis=(0, 1))
    inv_area = 1.0 / ((h // 2) * (w // 2))
    o_ref[...] = (s * inv_area).reshape(1, 1, cout).astype(o_ref.dtype)


def _conv_pool_gap(x_nchw, conv_w, conv_b):
    n, cin, h, w = x_nchw.shape
    cout = conv_w.shape[0]
    x16 = x_nchw.astype(jnp.bfloat16)
    w2 = jnp.transpose(conv_w, (2, 3, 1, 0)).reshape(9 * cin, cout)
    w2 = w2.astype(jnp.bfloat16)
    b2 = conv_b.reshape(1, cout)

    k = 9 * cin
    cost = pl.CostEstimate(
        flops=2 * n * h * w * k * cout,
        transcendentals=0,
        bytes_accessed=2 * n * h * w * cin + 4 * n * cout,
    )
    out = pl.pallas_call(
        functools.partial(_conv_pool_kernel, h=h, w=w, cout=cout),
        out_shape=jax.ShapeDtypeStruct((n, 1, cout), jnp.float32),
        grid=(n,),
        in_specs=[
            pl.BlockSpec((1, cin, h, w), lambda b: (b, 0, 0, 0)),
            pl.BlockSpec((k, cout), lambda b: (0, 0)),
            pl.BlockSpec((1, cout), lambda b: (0, 0)),
        ],
        out_specs=pl.BlockSpec((1, 1, cout), lambda b: (b, 0, 0)),
        compiler_params=pltpu.CompilerParams(
            dimension_semantics=("parallel",)
        ),
        cost_estimate=cost,
    )(x16, w2, b2)
    return out.reshape(n, cout)


def _mlp_kernel(x_ref, w1_ref, b1_ref, w2_ref, b2_ref, w3_ref, b3_ref, o_ref):
    h1 = jnp.dot(x_ref[...], w1_ref[...], preferred_element_type=jnp.float32)
    h1 = jnp.maximum(h1 + b1_ref[...], 0.0).astype(jnp.bfloat16)
    h2 = jnp.dot(h1, w2_ref[...], preferred_element_type=jnp.float32)
    h2 = jnp.maximum(h2 + b2_ref[...], 0.0).astype(jnp.bfloat16)
    h3 = jnp.dot(h2, w3_ref[...], preferred_element_type=jnp.float32)
    o_ref[...] = (h3 + b3_ref[...]).astype(o_ref.dtype)


def _classifier(x, fc1_w, fc1_b, fc2_w, fc2_b, fc3_w, fc3_b):
    m, d = x.shape
    nout = fc3_w.shape[1]
    mb = m // 2
    return pl.pallas_call(
        _mlp_kernel,
        out_shape=jax.ShapeDtypeStruct((m, nout), jnp.float32),
        grid=(2,),
        in_specs=[
            pl.BlockSpec((mb, d), lambda i: (i, 0)),
            pl.BlockSpec((d, d), lambda i: (0, 0)),
            pl.BlockSpec((1, d), lambda i: (0, 0)),
            pl.BlockSpec((d, d), lambda i: (0, 0)),
            pl.BlockSpec((1, d), lambda i: (0, 0)),
            pl.BlockSpec((d, nout), lambda i: (0, 0)),
            pl.BlockSpec((1, nout), lambda i: (0, 0)),
        ],
        out_specs=pl.BlockSpec((mb, nout), lambda i: (i, 0)),
        compiler_params=pltpu.CompilerParams(
            dimension_semantics=("parallel",)
        ),
    )(
        x.astype(jnp.bfloat16),
        fc1_w.astype(jnp.bfloat16), fc1_b.reshape(1, -1),
        fc2_w.astype(jnp.bfloat16), fc2_b.reshape(1, -1),
        fc3_w.astype(jnp.bfloat16), fc3_b.reshape(1, -1),
    )


def kernel(x_nchw, conv_w, conv_b, fc1_w, fc1_b, fc2_w, fc2_b, fc3_w, fc3_b):
    pooled = _conv_pool_gap(x_nchw, conv_w, conv_b)
    return _classifier(pooled, fc1_w, fc1_b, fc2_w, fc2_b, fc3_w, fc3_b)


# speedup vs baseline: 2.7806x; 1.2967x over previous
import functools

import jax
import jax.numpy as jnp
from jax.experimental import pallas as pl
from jax.experimental.pallas import tpu as pltpu


def _conv_pool_kernel(x_ref, w_ref, o_ref, *, h, w, cout, img):
    inv_area = 1.0 / ((h // 2) * (w // 2))
    for i in range(img):
        xt = jnp.transpose(x_ref[i], (1, 2, 0))
        xb = jnp.pad(xt, ((1, 1), (1, 1), (0, 0)))
        wide = jnp.concatenate(
            [xb[:, dx:dx + w, :] for dx in range(3)], axis=2)
        patches = jnp.concatenate(
            [wide[dy:dy + h] for dy in range(3)], axis=2)
        p2 = patches.reshape(h * w, 9 * 3)
        p2 = jnp.concatenate(
            [p2, jnp.ones((h * w, 1), jnp.bfloat16)], axis=1)
        y = jnp.dot(p2, w_ref[...], preferred_element_type=jnp.float32)
        y3 = y.reshape(h // 2, 2 * w, cout)
        rm = jnp.maximum(y3[:, :w, :], y3[:, w:, :])
        pm = jnp.maximum(rm[:, :w - 1, :], rm[:, 1:, :])
        pr = jnp.maximum(pm, 0.0)
        sg = jnp.sum(pr, axis=0)
        col = jax.lax.broadcasted_iota(jnp.int32, (w - 1, 1), 0)
        s = jnp.sum(jnp.where(col % 2 == 0, sg, 0.0), axis=0)
        o_ref[i] = (s * inv_area).reshape(1, cout).astype(o_ref.dtype)


def _conv_pool_gap(x_nchw, conv_w, conv_b):
    n, cin, h, w = x_nchw.shape
    cout = conv_w.shape[0]
    x16 = x_nchw.astype(jnp.bfloat16)
    w2 = jnp.transpose(conv_w, (2, 3, 1, 0)).reshape(9 * cin, cout)
    w_aug = jnp.concatenate([w2, conv_b.reshape(1, cout)], axis=0)
    w_aug = w_aug.astype(jnp.bfloat16)

    k = 9 * cin + 1
    img = 4
    cost = pl.CostEstimate(
        flops=2 * n * h * w * k * cout,
        transcendentals=0,
        bytes_accessed=2 * n * h * w * cin + 4 * n * cout,
    )
    out = pl.pallas_call(
        functools.partial(_conv_pool_kernel, h=h, w=w, cout=cout, img=img),
        out_shape=jax.ShapeDtypeStruct((n, 1, cout), jnp.float32),
        grid=(n // img,),
        in_specs=[
            pl.BlockSpec((img, cin, h, w), lambda b: (b, 0, 0, 0)),
            pl.BlockSpec((k, cout), lambda b: (0, 0)),
        ],
        out_specs=pl.BlockSpec((img, 1, cout), lambda b: (b, 0, 0)),
        compiler_params=pltpu.CompilerParams(
            dimension_semantics=("parallel",)
        ),
        cost_estimate=cost,
    )(x16, w_aug)
    return out.reshape(n, cout)


def _mlp_kernel(x_ref, w1_ref, b1_ref, w2_ref, b2_ref, w3_ref, b3_ref, o_ref):
    h1 = jnp.dot(x_ref[...], w1_ref[...], preferred_element_type=jnp.float32)
    h1 = jnp.maximum(h1 + b1_ref[...], 0.0).astype(jnp.bfloat16)
    h2 = jnp.dot(h1, w2_ref[...], preferred_element_type=jnp.float32)
    h2 = jnp.maximum(h2 + b2_ref[...], 0.0).astype(jnp.bfloat16)
    h3 = jnp.dot(h2, w3_ref[...], preferred_element_type=jnp.float32)
    o_ref[...] = (h3 + b3_ref[...]).astype(o_ref.dtype)


def _classifier(x, fc1_w, fc1_b, fc2_w, fc2_b, fc3_w, fc3_b):
    m, d = x.shape
    nout = fc3_w.shape[1]
    mb = m // 2
    return pl.pallas_call(
        _mlp_kernel,
        out_shape=jax.ShapeDtypeStruct((m, nout), jnp.float32),
        grid=(2,),
        in_specs=[
            pl.BlockSpec((mb, d), lambda i: (i, 0)),
            pl.BlockSpec((d, d), lambda i: (0, 0)),
            pl.BlockSpec((1, d), lambda i: (0, 0)),
            pl.BlockSpec((d, d), lambda i: (0, 0)),
            pl.BlockSpec((1, d), lambda i: (0, 0)),
            pl.BlockSpec((d, nout), lambda i: (0, 0)),
            pl.BlockSpec((1, nout), lambda i: (0, 0)),
        ],
        out_specs=pl.BlockSpec((mb, nout), lambda i: (i, 0)),
        compiler_params=pltpu.CompilerParams(
            dimension_semantics=("parallel",)
        ),
    )(
        x.astype(jnp.bfloat16),
        fc1_w.astype(jnp.bfloat16), fc1_b.reshape(1, -1),
        fc2_w.astype(jnp.bfloat16), fc2_b.reshape(1, -1),
        fc3_w.astype(jnp.bfloat16), fc3_b.reshape(1, -1),
    )


def kernel(x_nchw, conv_w, conv_b, fc1_w, fc1_b, fc2_w, fc2_b, fc3_w, fc3_b):
    pooled = _conv_pool_gap(x_nchw, conv_w, conv_b)
    return _classifier(pooled, fc1_w, fc1_b, fc2_w, fc2_b, fc3_w, fc3_b)
